# baseline (speedup 1.0000x reference)
import numpy as np
import jax
import jax.numpy as jnp
from jax.experimental import pallas as pl

_SIZE = 64
_HW = _SIZE * _SIZE
_P = _HW // 4
_K = 18
_KP = 32
_C = 256
_B = 16
_JBLK = 2048


def _mask_rows() -> np.ndarray:
    rng = np.random.default_rng(0)
    idx = rng.choice(_HW, _P, replace=False)
    m = np.zeros((_HW,), dtype=np.float32)
    m[idx] = 1.0
    return np.broadcast_to(m[None, :], (8, _HW)).copy()


_MASK8 = _mask_rows()


def _body(a_ref, p_ref, p2_ref, pt2_ref, m_ref, o_ref):
    f32 = jnp.float32
    a = a_ref[0]
    protos = p_ref[0]
    p2 = p2_ref[0]
    pt2 = pt2_ref[0]

    a_hi = a.astype(jnp.bfloat16)
    a_lo = (a - a_hi.astype(f32)).astype(jnp.bfloat16)

    s_h = jnp.dot(p2, a_hi, preferred_element_type=f32)
    s_l = jnp.dot(p2, a_lo, preferred_element_type=f32)
    s = (s_h[0:_KP, :] + s_h[_KP:2 * _KP, :]) + \
        (s_l[0:_KP, :] + s_l[_KP:2 * _KP, :])

    norms = jnp.sum(protos * protos, axis=1, keepdims=True)
    d = norms - 2.0 * s[0:_K, :]

    best_v = d[0:1, :]
    best_i = jnp.zeros((1, _JBLK), dtype=jnp.int32)
    for k in range(1, _K):
        row = d[k:k + 1, :]
        take = row < best_v
        best_v = jnp.where(take, row, best_v)
        best_i = jnp.where(take, jnp.int32(k), best_i)

    iota2 = jax.lax.broadcasted_iota(jnp.int32, (2 * _KP, _JBLK), 0)
    iota_mod = jnp.where(iota2 >= _KP, iota2 - _KP, iota2)
    onehot2 = (iota_mod == best_i).astype(jnp.bfloat16)
    sel = jnp.dot(pt2, onehot2, preferred_element_type=f32)

    m = m_ref[0:1, :]
    o_ref[0] = jnp.where(m != 0.0, sel, a)


def kernel(assp_features, prototypes):
    f32 = jnp.float32
    a3 = assp_features.reshape(_B, _C, _HW)
    p_hi = prototypes.astype(jnp.bfloat16)
    p_lo = (prototypes - p_hi.astype(f32)).astype(jnp.bfloat16)
    zpad = jnp.zeros((_B, _KP - _K, _C), dtype=jnp.bfloat16)
    p2 = jnp.concatenate([p_hi, zpad, p_lo, zpad], axis=1)
    pt2 = jnp.transpose(p2, (0, 2, 1))
    mask8 = jnp.asarray(_MASK8)

    grid = (_B, _HW // _JBLK)
    out = pl.pallas_call(
        _body,
        grid=grid,
        in_specs=[
            pl.BlockSpec((1, _C, _JBLK), lambda b, j: (b, 0, j)),
            pl.BlockSpec((1, _K, _C), lambda b, j: (b, 0, 0)),
            pl.BlockSpec((1, 2 * _KP, _C), lambda b, j: (b, 0, 0)),
            pl.BlockSpec((1, _C, 2 * _KP), lambda b, j: (b, 0, 0)),
            pl.BlockSpec((8, _JBLK), lambda b, j: (0, j)),
        ],
        out_specs=pl.BlockSpec((1, _C, _JBLK), lambda b, j: (b, 0, j)),
        out_shape=jax.ShapeDtypeStruct((_B, _C, _HW), jnp.float32),
    )(a3, prototypes, p2, pt2, mask8)
    return out.reshape(_B, _C, _SIZE, _SIZE)


# speedup vs baseline: 2.0757x; 2.0757x over previous
import numpy as np
import jax
import jax.numpy as jnp
from jax.experimental import pallas as pl

_SIZE = 64
_HW = _SIZE * _SIZE
_P = _HW // 4
_K = 18
_KP = 32
_C = 256
_B = 16
_JBLK = 2048
_BIG = 1e30


def _mask_col() -> np.ndarray:
    rng = np.random.default_rng(0)
    idx = rng.choice(_HW, _P, replace=False)
    m = np.zeros((_HW, 1), dtype=np.float32)
    m[idx, 0] = 1.0
    return m


_MASK = _mask_col()


def _body(a_ref, ptf_ref, pth_ref, ptl_ref, p4_ref, m_ref, o_ref):
    f32 = jnp.float32
    bf16 = jnp.bfloat16
    a = a_ref[0]
    ptf = ptf_ref[0]
    pth = pth_ref[0]
    ptl = ptl_ref[0]
    p4 = p4_ref[0]

    sq = ptf * ptf
    sq_hi = sq.astype(bf16)
    sq_lo = (sq - sq_hi.astype(f32)).astype(bf16)
    ones8 = jnp.ones((8, _C), dtype=bf16)
    norms8 = (jnp.dot(ones8, sq_hi, preferred_element_type=f32)
              + jnp.dot(ones8, sq_lo, preferred_element_type=f32))
    norms = norms8[0:1, :]

    a_hi = a.astype(bf16)
    a_lo = (a - a_hi.astype(f32)).astype(bf16)

    s = ((jnp.dot(a_hi, pth, preferred_element_type=f32)
          + jnp.dot(a_lo, pth, preferred_element_type=f32))
         + (jnp.dot(a_hi, ptl, preferred_element_type=f32)
            + jnp.dot(a_lo, ptl, preferred_element_type=f32)))

    iota_k = jax.lax.broadcasted_iota(jnp.int32, (_JBLK, _KP), 1)
    d = norms - 2.0 * s
    d = jnp.where(iota_k >= _K, jnp.full_like(d, _BIG), d)

    best = jnp.min(d, axis=1, keepdims=True)
    kst = jnp.min(jnp.where(d == best, iota_k, jnp.int32(_KP)),
                  axis=1, keepdims=True)

    iota2 = jax.lax.broadcasted_iota(jnp.int32, (_JBLK, 2 * _KP), 1)
    iota_mod = jnp.where(iota2 >= _KP, iota2 - _KP, iota2)
    onehot2 = (iota_mod == kst).astype(bf16)
    sel = jnp.dot(onehot2, p4, preferred_element_type=f32)

    m = m_ref[...]
    o_ref[0] = jnp.where(m != 0.0, sel, a)


def kernel(assp_features, prototypes):
    f32 = jnp.float32
    bf16 = jnp.bfloat16
    at = jnp.transpose(assp_features, (0, 2, 3, 1)).reshape(_B, _HW, _C)

    ptf = jnp.concatenate(
        [jnp.transpose(prototypes, (0, 2, 1)),
         jnp.zeros((_B, _C, _KP - _K), dtype=f32)], axis=2)
    pth = ptf.astype(bf16)
    ptl = (ptf - pth.astype(f32)).astype(bf16)
    p_hi = jnp.transpose(pth, (0, 2, 1))
    p_lo = jnp.transpose(ptl, (0, 2, 1))
    p4 = jnp.concatenate([p_hi, p_lo], axis=1)
    mask = jnp.asarray(_MASK)

    grid = (_B, _HW // _JBLK)
    out = pl.pallas_call(
        _body,
        grid=grid,
        in_specs=[
            pl.BlockSpec((1, _JBLK, _C), lambda b, j: (b, j, 0)),
            pl.BlockSpec((1, _C, _KP), lambda b, j: (b, 0, 0)),
            pl.BlockSpec((1, _C, _KP), lambda b, j: (b, 0, 0)),
            pl.BlockSpec((1, _C, _KP), lambda b, j: (b, 0, 0)),
            pl.BlockSpec((1, 2 * _KP, _C), lambda b, j: (b, 0, 0)),
            pl.BlockSpec((_JBLK, 1), lambda b, j: (j, 0)),
        ],
        out_specs=pl.BlockSpec((1, _JBLK, _C), lambda b, j: (b, j, 0)),
        out_shape=jax.ShapeDtypeStruct((_B, _HW, _C), jnp.float32),
    )(at, ptf, pth, ptl, p4, mask)
    return jnp.transpose(out.reshape(_B, _SIZE, _SIZE, _C), (0, 3, 1, 2))


# speedup vs baseline: 2.0941x; 1.0089x over previous
import numpy as np
import jax
import jax.numpy as jnp
from jax.experimental import pallas as pl

_SIZE = 64
_HW = _SIZE * _SIZE
_P = _HW // 4
_K = 18
_KP = 32
_C = 256
_B = 16
_JBLK = 2048
_BIG = 1e30


def _mask_col() -> np.ndarray:
    rng = np.random.default_rng(0)
    idx = rng.choice(_HW, _P, replace=False)
    m = np.zeros((_HW, 1), dtype=np.float32)
    m[idx, 0] = 1.0
    return m


_MASK = _mask_col()


def _body(a_ref, ptf_ref, pkc_ref, m_ref, o_ref):
    f32 = jnp.float32
    bf16 = jnp.bfloat16
    a = a_ref[0]
    ptf = ptf_ref[0]
    pkc = pkc_ref[0]

    pth = ptf.astype(bf16)
    ptl = (ptf - pth.astype(f32)).astype(bf16)
    pt4 = jnp.concatenate([pth, ptl, ptl, pth], axis=0)
    pkh = pkc.astype(bf16)
    pkl = (pkc - pkh.astype(f32)).astype(bf16)
    p4 = jnp.concatenate([pkh, pkl], axis=0)

    sq = ptf * ptf
    sq_hi = sq.astype(bf16)
    sq_lo = (sq - sq_hi.astype(f32)).astype(bf16)
    sq2 = jnp.concatenate([sq_hi, sq_lo], axis=0)
    ones8 = jnp.ones((8, 2 * _C), dtype=bf16)
    norms8 = jnp.dot(ones8, sq2, preferred_element_type=f32)
    norms = norms8[0:1, :]

    a_hi = a.astype(bf16)
    a_lo = (a - a_hi.astype(f32)).astype(bf16)
    a4 = jnp.concatenate([a_hi, a_lo, a_hi, a_lo], axis=1)

    s = jnp.dot(a4, pt4, preferred_element_type=f32)

    iota_k = jax.lax.broadcasted_iota(jnp.int32, (_JBLK, _KP), 1)
    d = norms - 2.0 * s
    d = jnp.where(iota_k >= _K, jnp.full_like(d, _BIG), d)

    best = jnp.min(d, axis=1, keepdims=True)
    kst = jnp.min(jnp.where(d == best, iota_k, jnp.int32(_KP)),
                  axis=1, keepdims=True)

    iota2 = jax.lax.broadcasted_iota(jnp.int32, (_JBLK, 2 * _KP), 1)
    iota_mod = jnp.where(iota2 >= _KP, iota2 - _KP, iota2)
    onehot2 = (iota_mod == kst).astype(bf16)
    sel = jnp.dot(onehot2, p4, preferred_element_type=f32)

    m = m_ref[...]
    o_ref[0] = jnp.where(m != 0.0, sel, a)


def kernel(assp_features, prototypes):
    f32 = jnp.float32
    bf16 = jnp.bfloat16
    at = jnp.transpose(assp_features, (0, 2, 3, 1)).reshape(_B, _HW, _C)

    ptf = jnp.concatenate(
        [jnp.transpose(prototypes, (0, 2, 1)),
         jnp.zeros((_B, _C, _KP - _K), dtype=f32)], axis=2)
    pkc = jnp.concatenate(
        [prototypes,
         jnp.zeros((_B, _KP - _K, _C), dtype=f32)], axis=1)
    mask = jnp.asarray(_MASK)

    grid = (_B, _HW // _JBLK)
    out = pl.pallas_call(
        _body,
        grid=grid,
        in_specs=[
            pl.BlockSpec((1, _JBLK, _C), lambda b, j: (b, j, 0)),
            pl.BlockSpec((1, _C, _KP), lambda b, j: (b, 0, 0)),
            pl.BlockSpec((1, _KP, _C), lambda b, j: (b, 0, 0)),
            pl.BlockSpec((_JBLK, 1), lambda b, j: (j, 0)),
        ],
        out_specs=pl.BlockSpec((1, _JBLK, _C), lambda b, j: (b, j, 0)),
        out_shape=jax.ShapeDtypeStruct((_B, _HW, _C), jnp.float32),
    )(at, ptf, pkc, mask)
    return jnp.transpose(out.reshape(_B, _SIZE, _SIZE, _C), (0, 3, 1, 2))


# speedup vs baseline: 2.5398x; 1.2128x over previous
import numpy as np
import jax
import jax.numpy as jnp
from jax.experimental import pallas as pl

_SIZE = 64
_HW = _SIZE * _SIZE
_P = _HW // 4
_K = 18
_KP = 32
_C = 256
_B = 16
_JBLK = 4096
_BIG = 1e30


def _mask_col() -> np.ndarray:
    rng = np.random.default_rng(0)
    idx = rng.choice(_HW, _P, replace=False)
    m = np.zeros((_HW, 1), dtype=np.float32)
    m[idx, 0] = 1.0
    return m


_MASK = _mask_col()


def _body(a_ref, ptf_ref, pkc_ref, m_ref, o_ref):
    f32 = jnp.float32
    bf16 = jnp.bfloat16
    a = a_ref[0]
    ptf = ptf_ref[0]
    pkc = pkc_ref[0]

    pth = ptf.astype(bf16)
    ptl = (ptf - pth.astype(f32)).astype(bf16)
    pt4 = jnp.concatenate([pth, ptl, ptl, pth], axis=0)
    pkh = pkc.astype(bf16)
    pkl = (pkc - pkh.astype(f32)).astype(bf16)
    p4 = jnp.concatenate([pkh, pkl], axis=0)

    sq = ptf * ptf
    sq_hi = sq.astype(bf16)
    sq_lo = (sq - sq_hi.astype(f32)).astype(bf16)
    sq2 = jnp.concatenate([sq_hi, sq_lo], axis=0)
    ones8 = jnp.ones((8, 2 * _C), dtype=bf16)
    norms8 = jnp.dot(ones8, sq2, preferred_element_type=f32)
    norms = norms8[0:1, :]

    a_hi = a.astype(bf16)
    a_lo = (a - a_hi.astype(f32)).astype(bf16)
    a4 = jnp.concatenate([a_hi, a_lo, a_hi, a_lo], axis=1)

    s = jnp.dot(a4, pt4, preferred_element_type=f32)

    iota_k = jax.lax.broadcasted_iota(jnp.int32, (_JBLK, _KP), 1)
    d = norms - 2.0 * s
    d = jnp.where(iota_k >= _K, jnp.full_like(d, _BIG), d)

    best = jnp.min(d, axis=1, keepdims=True)
    kst = jnp.min(jnp.where(d == best, iota_k, jnp.int32(_KP)),
                  axis=1, keepdims=True)

    iota2 = jax.lax.broadcasted_iota(jnp.int32, (_JBLK, 2 * _KP), 1)
    iota_mod = jnp.where(iota2 >= _KP, iota2 - _KP, iota2)
    onehot2 = (iota_mod == kst).astype(bf16)
    sel = jnp.dot(onehot2, p4, preferred_element_type=f32)

    m = m_ref[...]
    o_ref[0] = jnp.where(m != 0.0, sel, a)


def kernel(assp_features, prototypes):
    f32 = jnp.float32
    bf16 = jnp.bfloat16
    at = jnp.transpose(assp_features, (0, 2, 3, 1)).reshape(_B, _HW, _C)

    ptf = jnp.concatenate(
        [jnp.transpose(prototypes, (0, 2, 1)),
         jnp.zeros((_B, _C, _KP - _K), dtype=f32)], axis=2)
    pkc = jnp.concatenate(
        [prototypes,
         jnp.zeros((_B, _KP - _K, _C), dtype=f32)], axis=1)
    mask = jnp.asarray(_MASK)

    grid = (_B,)
    out = pl.pallas_call(
        _body,
        grid=grid,
        in_specs=[
            pl.BlockSpec((1, _JBLK, _C), lambda b: (b, 0, 0)),
            pl.BlockSpec((1, _C, _KP), lambda b: (b, 0, 0)),
            pl.BlockSpec((1, _KP, _C), lambda b: (b, 0, 0)),
            pl.BlockSpec((_JBLK, 1), lambda b: (0, 0)),
        ],
        out_specs=pl.BlockSpec((1, _JBLK, _C), lambda b: (b, 0, 0)),
        out_shape=jax.ShapeDtypeStruct((_B, _HW, _C), jnp.float32),
    )(at, ptf, pkc, mask)
    return jnp.transpose(out.reshape(_B, _SIZE, _SIZE, _C), (0, 3, 1, 2))
